# baseline (speedup 1.0000x reference)
import jax
import jax.numpy as jnp
from jax import lax
from jax.experimental import pallas as pl
from jax.experimental.pallas import tpu as pltpu


def _gather_outer_kernel(S, BB, NR, fil_hbm, rol_ref, ft_hbm, rt_ref, x3_ref,
                         ft_vmem, feA, feB, reA, reB, pbA, pbB, p0, p1, p2, p3,
                         ft_sem, sem0, sem1, sem2, sem3):
    i = pl.program_id(0)

    @pl.when(i == 0)
    def _load_table():
        cp = pltpu.make_async_copy(ft_hbm, ft_vmem, ft_sem)
        cp.start()
        cp.wait()

    row0 = i * BB
    bufs = (p0, p1, p2, p3)
    sems = (sem0, sem1, sem2, sem3)

    for k in range(4):
        pltpu.make_async_copy(fil_hbm.at[row0 + k], bufs[k], sems[k]).start()

    iota8 = lax.broadcasted_iota(jnp.int32, (8, S), 0)

    def gather_b(p_sm, fe_t):
        for s in range(S):
            fe_t[s] = ft_vmem[p_sm[s], 0]

    def role_onehot_b(bl, pb_t, re_t):
        base = pl.multiple_of((bl >> 3) << 3, 8)
        chunk = rol_ref[pl.ds(base, 8), :]
        rvec = pltpu.roll(chunk, -(bl & 7), axis=0)[0:1, :]
        d = jnp.broadcast_to(rvec, (8, S)) - iota8
        for t in range(NR // 8):
            pb_t[8 * t:8 * (t + 1), :] = jnp.where(d == 8 * t, 1.0, 0.0)
        re_t[...] = lax.dot_general(
            rt_ref[...], pb_t[...].astype(jnp.bfloat16),
            (((0,), (0,)), ((), ())),
            preferred_element_type=jnp.float32).astype(jnp.bfloat16)

    def dot_b(fe_t, re_t):
        return lax.dot_general(fe_t[...].astype(jnp.bfloat16), re_t[...],
                               (((0,), (1,)), ((), ())),
                               preferred_element_type=jnp.float32
                               ).astype(jnp.bfloat16)

    def body(j, carry):
        b0 = 4 * j
        for k in range(4):
            bl = b0 + k
            fe_c, re_c, pb_c = (feA, reA, pbA) if k % 2 == 0 else (feB, reB, pbB)
            fe_p, re_p = (feB, reB) if k % 2 == 0 else (feA, reA)
            pltpu.make_async_copy(fil_hbm.at[row0 + bl], bufs[k], sems[k]).wait()
            o = dot_b(fe_p, re_p)
            gather_b(bufs[k], fe_c)
            role_onehot_b(bl, pb_c, re_c)
            x3_ref[jnp.maximum(bl - 1, 0)] = o
            nxt = jnp.minimum(row0 + bl + 4, row0 + BB - 1)
            pltpu.make_async_copy(fil_hbm.at[nxt], bufs[k], sems[k]).start()
        return carry

    lax.fori_loop(0, BB // 4, body, 0)
    for k in range(4):
        pltpu.make_async_copy(fil_hbm.at[row0 + BB - 1], bufs[k], sems[k]).wait()
    x3_ref[BB - 1] = dot_b(feB, reB)


def _final_matmul_kernel(MB, FW, x_ref, w_ref, b_ref, o_ref):
    t = pl.program_id(1)

    @pl.when(t == 0)
    def _init():
        o_ref[...] = jnp.broadcast_to(b_ref[...], o_ref.shape)

    d = lax.dot_general(x_ref[...], w_ref[...], (((1,), (1,)), ((), ())),
                        preferred_element_type=jnp.float32)
    o_ref[...] = o_ref[...] + d


def kernel(fillers, roles, filler_table, role_table, W_last, b_last):
    B, S = fillers.shape
    NF, FD = filler_table.shape
    NR, RD = role_table.shape
    FW = W_last.shape[0]

    NB = 4
    BB = B // NB

    ft3 = filler_table.reshape(NF, 1, FD)
    fillers = fillers.astype(jnp.int32)
    roles = roles.astype(jnp.int32)

    x3 = pl.pallas_call(
        lambda *a: _gather_outer_kernel(S, BB, NR, *a),
        out_shape=jax.ShapeDtypeStruct((B, FD, RD), jnp.bfloat16),
        grid=(NB,),
        in_specs=[
            pl.BlockSpec(memory_space=pl.ANY),
            pl.BlockSpec((BB, S), lambda i: (i, 0)),
            pl.BlockSpec(memory_space=pl.ANY),
            pl.BlockSpec((NR, RD), lambda i: (0, 0)),
        ],
        out_specs=pl.BlockSpec((BB, FD, RD), lambda i: (i, 0, 0)),
        scratch_shapes=[
            pltpu.VMEM((NF, 1, FD), jnp.float32),
            pltpu.VMEM((S, FD), jnp.float32),
            pltpu.VMEM((S, FD), jnp.float32),
            pltpu.VMEM((RD, S), jnp.bfloat16),
            pltpu.VMEM((RD, S), jnp.bfloat16),
            pltpu.VMEM((NR, S), jnp.float32),
            pltpu.VMEM((NR, S), jnp.float32),
            pltpu.SMEM((S,), jnp.int32),
            pltpu.SMEM((S,), jnp.int32),
            pltpu.SMEM((S,), jnp.int32),
            pltpu.SMEM((S,), jnp.int32),
            pltpu.SemaphoreType.DMA,
            pltpu.SemaphoreType.DMA,
            pltpu.SemaphoreType.DMA,
            pltpu.SemaphoreType.DMA,
            pltpu.SemaphoreType.DMA,
        ],
        compiler_params=pltpu.CompilerParams(
            dimension_semantics=("arbitrary",),
            vmem_limit_bytes=52 * 1024 * 1024,
        ),
        name="gather_outer",
    )(fillers, roles, ft3, role_table.astype(jnp.bfloat16))

    MB = B // 2
    KB = min(1024, FD * RD)
    NT = (FD * RD) // KB
    x2 = x3.reshape(B, FD * RD)
    out2 = pl.pallas_call(
        lambda *a: _final_matmul_kernel(MB, FW, *a),
        out_shape=jax.ShapeDtypeStruct((B, FW), jnp.float32),
        grid=(2, NT),
        in_specs=[
            pl.BlockSpec((MB, KB), lambda m, t: (m, t)),
            pl.BlockSpec((FW, KB), lambda m, t: (0, t)),
            pl.BlockSpec((1, FW), lambda m, t: (0, 0)),
        ],
        out_specs=pl.BlockSpec((MB, FW), lambda m, t: (m, 0)),
        compiler_params=pltpu.CompilerParams(
            dimension_semantics=("arbitrary", "arbitrary"),
            vmem_limit_bytes=48 * 1024 * 1024,
        ),
        name="final_linear",
    )(x2, W_last.astype(jnp.bfloat16), b_last.reshape(1, FW))

    return out2[None]


# speedup vs baseline: 1.0321x; 1.0321x over previous
import jax
import jax.numpy as jnp
from jax import lax
from jax.experimental import pallas as pl
from jax.experimental.pallas import tpu as pltpu


def _gather_outer_kernel(S, BB, NR, fil_hbm, rol_ref, ft_hbm, rt_ref, x3_ref,
                         ft_vmem, feA, feB, reA, reB, pbA, pbB, p0, p1, p2, p3,
                         ft_sem, sem0, sem1, sem2, sem3):
    i = pl.program_id(0)

    @pl.when(i == 0)
    def _load_table():
        cp = pltpu.make_async_copy(ft_hbm, ft_vmem, ft_sem)
        cp.start()
        cp.wait()

    row0 = i * BB
    bufs = (p0, p1, p2, p3)
    sems = (sem0, sem1, sem2, sem3)

    for k in range(4):
        pltpu.make_async_copy(fil_hbm.at[row0 + k], bufs[k], sems[k]).start()

    iota8 = lax.broadcasted_iota(jnp.int32, (8, S), 0)

    def gather_b(p_sm, fe_t):
        for s in range(S):
            fe_t[s] = ft_vmem[p_sm[s], 0]

    def role_onehot_b(bl, pb_t, re_t):
        base = pl.multiple_of((bl >> 3) << 3, 8)
        chunk = rol_ref[pl.ds(base, 8), :]
        rvec = pltpu.roll(chunk, -(bl & 7), axis=0)[0:1, :]
        d = jnp.broadcast_to(rvec, (8, S)) - iota8
        for t in range(NR // 8):
            pb_t[8 * t:8 * (t + 1), :] = jnp.where(d == 8 * t, 1.0, 0.0)
        re_t[...] = lax.dot_general(
            rt_ref[...], pb_t[...].astype(jnp.bfloat16),
            (((0,), (0,)), ((), ())),
            preferred_element_type=jnp.float32).astype(jnp.bfloat16)

    def dot_b(fe_t, re_t):
        return lax.dot_general(fe_t[...].astype(jnp.bfloat16), re_t[...],
                               (((0,), (1,)), ((), ())),
                               preferred_element_type=jnp.float32)

    def body(j, carry):
        b0 = 8 * j
        for k in range(8):
            bl = b0 + k
            fe_c, re_c, pb_c = (feA, reA, pbA) if k % 2 == 0 else (feB, reB, pbB)
            fe_p, re_p = (feB, reB) if k % 2 == 0 else (feA, reA)
            pltpu.make_async_copy(fil_hbm.at[row0 + bl], bufs[k % 4],
                                  sems[k % 4]).wait()
            o = dot_b(fe_p, re_p)
            gather_b(bufs[k % 4], fe_c)
            role_onehot_b(bl, pb_c, re_c)
            x3_ref[jnp.maximum(bl - 1, 0)] = o
            nxt = jnp.minimum(row0 + bl + 4, row0 + BB - 1)
            pltpu.make_async_copy(fil_hbm.at[nxt], bufs[k % 4], sems[k % 4]).start()
        return carry

    lax.fori_loop(0, BB // 8, body, 0)
    for k in range(4):
        pltpu.make_async_copy(fil_hbm.at[row0 + BB - 1], bufs[k], sems[k]).wait()
    x3_ref[BB - 1] = dot_b(feB, reB)


def _final_matmul_kernel(MB, FW, x_ref, w_ref, b_ref, o_ref):
    t = pl.program_id(1)

    @pl.when(t == 0)
    def _init():
        o_ref[...] = jnp.broadcast_to(b_ref[...], o_ref.shape)

    acc = None
    RD = x_ref.shape[2]
    for j in range(x_ref.shape[1]):
        xj = x_ref[:, j, :]
        wj = w_ref[:, j * RD:(j + 1) * RD]
        d = lax.dot_general(xj, wj, (((1,), (1,)), ((), ())),
                            preferred_element_type=jnp.float32)
        acc = d if acc is None else acc + d
    o_ref[...] = o_ref[...] + acc


def kernel(fillers, roles, filler_table, role_table, W_last, b_last):
    B, S = fillers.shape
    NF, FD = filler_table.shape
    NR, RD = role_table.shape
    FW = W_last.shape[0]

    NB = 4
    BB = B // NB

    ft3 = filler_table.reshape(NF, 1, FD)
    fillers = fillers.astype(jnp.int32)
    roles = roles.astype(jnp.int32)

    x3 = pl.pallas_call(
        lambda *a: _gather_outer_kernel(S, BB, NR, *a),
        out_shape=jax.ShapeDtypeStruct((B, FD, RD), jnp.float32),
        grid=(NB,),
        in_specs=[
            pl.BlockSpec(memory_space=pl.ANY),
            pl.BlockSpec((BB, S), lambda i: (i, 0)),
            pl.BlockSpec(memory_space=pl.ANY),
            pl.BlockSpec((NR, RD), lambda i: (0, 0)),
        ],
        out_specs=pl.BlockSpec((BB, FD, RD), lambda i: (i, 0, 0)),
        scratch_shapes=[
            pltpu.VMEM((NF, 1, FD), jnp.float32),
            pltpu.VMEM((S, FD), jnp.float32),
            pltpu.VMEM((S, FD), jnp.float32),
            pltpu.VMEM((RD, S), jnp.bfloat16),
            pltpu.VMEM((RD, S), jnp.bfloat16),
            pltpu.VMEM((NR, S), jnp.float32),
            pltpu.VMEM((NR, S), jnp.float32),
            pltpu.SMEM((S,), jnp.int32),
            pltpu.SMEM((S,), jnp.int32),
            pltpu.SMEM((S,), jnp.int32),
            pltpu.SMEM((S,), jnp.int32),
            pltpu.SemaphoreType.DMA,
            pltpu.SemaphoreType.DMA,
            pltpu.SemaphoreType.DMA,
            pltpu.SemaphoreType.DMA,
            pltpu.SemaphoreType.DMA,
        ],
        compiler_params=pltpu.CompilerParams(
            dimension_semantics=("arbitrary",),
            vmem_limit_bytes=52 * 1024 * 1024,
        ),
        name="gather_outer",
    )(fillers, roles, ft3, role_table.astype(jnp.bfloat16))

    MB = B // 2
    FBLK = 8
    NT = FD // FBLK
    out2 = pl.pallas_call(
        lambda *a: _final_matmul_kernel(MB, FW, *a),
        out_shape=jax.ShapeDtypeStruct((B, FW), jnp.float32),
        grid=(2, NT),
        in_specs=[
            pl.BlockSpec((MB, FBLK, RD), lambda m, t: (m, t, 0)),
            pl.BlockSpec((FW, FBLK * RD), lambda m, t: (0, t)),
            pl.BlockSpec((1, FW), lambda m, t: (0, 0)),
        ],
        out_specs=pl.BlockSpec((MB, FW), lambda m, t: (m, 0)),
        compiler_params=pltpu.CompilerParams(
            dimension_semantics=("arbitrary", "arbitrary"),
            vmem_limit_bytes=48 * 1024 * 1024,
        ),
        name="final_linear",
    )(x3, W_last, b_last.reshape(1, FW))

    return out2[None]


# speedup vs baseline: 1.0345x; 1.0023x over previous
import jax
import jax.numpy as jnp
from jax import lax
from jax.experimental import pallas as pl
from jax.experimental.pallas import tpu as pltpu


def _gather_outer_kernel(S, BB, NR, fil_hbm, rol_ref, ft_hbm, rt_ref, x3_ref,
                         ft_vmem, feA, feB, reA, reB, pbA, pbB, p0, p1, p2, p3,
                         ft_sem, sem0, sem1, sem2, sem3):
    i = pl.program_id(0)

    @pl.when(i == 0)
    def _load_table():
        cp = pltpu.make_async_copy(ft_hbm, ft_vmem, ft_sem)
        cp.start()
        cp.wait()

    row0 = i * BB
    bufs = (p0, p1, p2, p3)
    sems = (sem0, sem1, sem2, sem3)

    for k in range(4):
        pltpu.make_async_copy(fil_hbm.at[row0 + k], bufs[k], sems[k]).start()

    iota8 = lax.broadcasted_iota(jnp.int32, (8, S), 0)

    def gather_b(p_sm, fe_t):
        for s in range(S):
            fe_t[s] = ft_vmem[p_sm[s], 0]

    def role_onehot_b(bl, pb_t, re_t):
        base = pl.multiple_of((bl >> 3) << 3, 8)
        chunk = rol_ref[pl.ds(base, 8), :]
        rvec = pltpu.roll(chunk, -(bl & 7), axis=0)[0:1, :]
        d = jnp.broadcast_to(rvec, (8, S)) - iota8
        for t in range(NR // 8):
            pb_t[8 * t:8 * (t + 1), :] = jnp.where(d == 8 * t, 1.0, 0.0)
        re_t[...] = lax.dot_general(
            rt_ref[...], pb_t[...].astype(jnp.bfloat16),
            (((0,), (0,)), ((), ())),
            preferred_element_type=jnp.float32).astype(jnp.bfloat16)

    def dot_b(fe_t, re_t):
        return lax.dot_general(fe_t[...].astype(jnp.bfloat16), re_t[...],
                               (((0,), (1,)), ((), ())),
                               preferred_element_type=jnp.float32)

    def body(j, carry):
        b0 = 4 * j
        for k in range(4):
            bl = b0 + k
            fe_c, re_c, pb_c = (feA, reA, pbA) if k % 2 == 0 else (feB, reB, pbB)
            fe_p, re_p = (feB, reB) if k % 2 == 0 else (feA, reA)
            pltpu.make_async_copy(fil_hbm.at[row0 + bl], bufs[k], sems[k]).wait()
            o = dot_b(fe_p, re_p)
            gather_b(bufs[k], fe_c)
            role_onehot_b(bl, pb_c, re_c)
            x3_ref[jnp.maximum(bl - 1, 0)] = o
            nxt = jnp.minimum(row0 + bl + 4, row0 + BB - 1)
            pltpu.make_async_copy(fil_hbm.at[nxt], bufs[k], sems[k]).start()
        return carry

    lax.fori_loop(0, BB // 4, body, 0)
    for k in range(4):
        pltpu.make_async_copy(fil_hbm.at[row0 + BB - 1], bufs[k], sems[k]).wait()
    x3_ref[BB - 1] = dot_b(feB, reB)


def _final_matmul_kernel(MB, FW, x_ref, w_ref, b_ref, o_ref):
    t = pl.program_id(1)

    @pl.when(t == 0)
    def _init():
        o_ref[...] = jnp.broadcast_to(b_ref[...], o_ref.shape)

    acc = None
    RD = x_ref.shape[2]
    for j in range(x_ref.shape[1]):
        xj = x_ref[:, j, :]
        wj = w_ref[:, j * RD:(j + 1) * RD]
        d = lax.dot_general(xj, wj, (((1,), (1,)), ((), ())),
                            preferred_element_type=jnp.float32)
        acc = d if acc is None else acc + d
    o_ref[...] = o_ref[...] + acc


def kernel(fillers, roles, filler_table, role_table, W_last, b_last):
    B, S = fillers.shape
    NF, FD = filler_table.shape
    NR, RD = role_table.shape
    FW = W_last.shape[0]

    NB = 4
    BB = B // NB

    ft3 = filler_table.reshape(NF, 1, FD)
    fillers = fillers.astype(jnp.int32)
    roles = roles.astype(jnp.int32)

    x3 = pl.pallas_call(
        lambda *a: _gather_outer_kernel(S, BB, NR, *a),
        out_shape=jax.ShapeDtypeStruct((B, FD, RD), jnp.float32),
        grid=(NB,),
        in_specs=[
            pl.BlockSpec(memory_space=pl.ANY),
            pl.BlockSpec((BB, S), lambda i: (i, 0)),
            pl.BlockSpec(memory_space=pl.ANY),
            pl.BlockSpec((NR, RD), lambda i: (0, 0)),
        ],
        out_specs=pl.BlockSpec((BB, FD, RD), lambda i: (i, 0, 0)),
        scratch_shapes=[
            pltpu.VMEM((NF, 1, FD), jnp.float32),
            pltpu.VMEM((S, FD), jnp.float32),
            pltpu.VMEM((S, FD), jnp.float32),
            pltpu.VMEM((RD, S), jnp.bfloat16),
            pltpu.VMEM((RD, S), jnp.bfloat16),
            pltpu.VMEM((NR, S), jnp.float32),
            pltpu.VMEM((NR, S), jnp.float32),
            pltpu.SMEM((S,), jnp.int32),
            pltpu.SMEM((S,), jnp.int32),
            pltpu.SMEM((S,), jnp.int32),
            pltpu.SMEM((S,), jnp.int32),
            pltpu.SemaphoreType.DMA,
            pltpu.SemaphoreType.DMA,
            pltpu.SemaphoreType.DMA,
            pltpu.SemaphoreType.DMA,
            pltpu.SemaphoreType.DMA,
        ],
        compiler_params=pltpu.CompilerParams(
            dimension_semantics=("arbitrary",),
            vmem_limit_bytes=52 * 1024 * 1024,
        ),
        name="gather_outer",
    )(fillers, roles, ft3, role_table.astype(jnp.bfloat16))

    MB = B // 2
    FBLK = 8
    NT = FD // FBLK
    out2 = pl.pallas_call(
        lambda *a: _final_matmul_kernel(MB, FW, *a),
        out_shape=jax.ShapeDtypeStruct((B, FW), jnp.float32),
        grid=(2, NT),
        in_specs=[
            pl.BlockSpec((MB, FBLK, RD), lambda m, t: (m, t, 0)),
            pl.BlockSpec((FW, FBLK * RD), lambda m, t: (0, t)),
            pl.BlockSpec((1, FW), lambda m, t: (0, 0)),
        ],
        out_specs=pl.BlockSpec((MB, FW), lambda m, t: (m, 0)),
        compiler_params=pltpu.CompilerParams(
            dimension_semantics=("arbitrary", "arbitrary"),
            vmem_limit_bytes=48 * 1024 * 1024,
        ),
        name="final_linear",
    )(x3, W_last, b_last.reshape(1, FW))

    return out2[None]
